# baseline (speedup 1.0000x reference)
import functools
import math

import jax
import jax.numpy as jnp
from jax import lax
from jax.experimental import pallas as pl
from jax.experimental.pallas import tpu as pltpu
from jax.experimental.pallas import tpu_sc as plsc

N = 10000
E = 320000
D = 128
D_EDGE = 16
POOLED = 8

NC = 2
NS = 16
NW = NC * NS
CH = 80
SUB_ROWS = 624
SUB_TAIL = N - NS * SUB_ROWS

PARTS = (192000, 128000)
POFF = (0, PARTS[0])

MLP_BLK = 4000


def _gelu(t):
    return 0.5 * t * (1.0 + lax.erf(t * (1.0 / math.sqrt(2.0))))


def _mm_bias_body(x_ref, w_ref, b_ref, o_ref):
    o_ref[...] = (
        jnp.dot(x_ref[...], w_ref[...], preferred_element_type=jnp.float32)
        + b_ref[...]
    )


def _mm_bias(x, w, b, blk):
    n, k = x.shape
    m = w.shape[1]
    return pl.pallas_call(
        _mm_bias_body,
        grid=(n // blk,),
        in_specs=[
            pl.BlockSpec((blk, k), lambda i: (i, 0)),
            pl.BlockSpec((k, m), lambda i: (0, 0)),
            pl.BlockSpec((1, m), lambda i: (0, 0)),
        ],
        out_specs=pl.BlockSpec((blk, m), lambda i: (i, 0)),
        out_shape=jax.ShapeDtypeStruct((n, m), jnp.float32),
    )(x, w, b.reshape(1, m))


def _edge_mlp_body(hs_ref, e_ref, w1_ref, b1_ref, o_ref):
    t = (
        jnp.dot(hs_ref[...] + e_ref[...], w1_ref[...],
                preferred_element_type=jnp.float32)
        + b1_ref[...]
    )
    o_ref[...] = _gelu(t)


def _edge_mlp(hs, e, w1, b1, e_off_blocks):
    ep = hs.shape[0]
    return pl.pallas_call(
        _edge_mlp_body,
        grid=(ep // MLP_BLK,),
        in_specs=[
            pl.BlockSpec((MLP_BLK, D), lambda i: (i, 0)),
            pl.BlockSpec((MLP_BLK, D), lambda i: (i + e_off_blocks, 0)),
            pl.BlockSpec((D, D), lambda i: (0, 0)),
            pl.BlockSpec((1, D), lambda i: (0, 0)),
        ],
        out_specs=pl.BlockSpec((MLP_BLK, D), lambda i: (i, 0)),
        out_shape=jax.ShapeDtypeStruct((ep, D), jnp.float32),
    )(hs, e, w1, b1.reshape(1, D))


def _deg_reduce_body(degp_ref, o_ref):
    deg = degp_ref[0, :, 0:1] + degp_ref[1, :, 0:1]
    o_ref[...] = jnp.broadcast_to(deg, (N, 8))


def _deg_reduce(degp):
    return pl.pallas_call(
        _deg_reduce_body,
        out_shape=jax.ShapeDtypeStruct((N, 8), jnp.float32),
    )(degp)


def _node_update_body(h_ref, agg_ref, agg2_ref, degp_ref, w2_ref, b2_ref,
                      u1w_ref, u1b_ref, u2w_ref, u2b_ref,
                      lnw_ref, lnb_ref, o_ref):
    agg1 = (agg_ref[0] + agg_ref[1]) + (agg2_ref[0] + agg2_ref[1])
    deg = degp_ref[:, 0:1]
    a2 = (
        jnp.dot(agg1, w2_ref[...], preferred_element_type=jnp.float32)
        + deg * b2_ref[...]
    )
    a2 = a2 / jnp.maximum(deg, 1.0)
    u = _gelu(
        jnp.dot(a2, u1w_ref[...], preferred_element_type=jnp.float32)
        + u1b_ref[...]
    )
    u = (
        jnp.dot(u, u2w_ref[...], preferred_element_type=jnp.float32)
        + u2b_ref[...]
    )
    hn = h_ref[...] + u
    mu = jnp.mean(hn, axis=1, keepdims=True)
    var = jnp.mean((hn - mu) ** 2, axis=1, keepdims=True)
    o_ref[...] = (hn - mu) / jnp.sqrt(var + 1e-5) * lnw_ref[...] + lnb_ref[...]


def _node_update(h, aggp, aggp2, degp, p, blk=2000):
    return pl.pallas_call(
        _node_update_body,
        grid=(N // blk,),
        in_specs=[
            pl.BlockSpec((blk, D), lambda i: (i, 0)),
            pl.BlockSpec((NC, blk, D), lambda i: (0, i, 0)),
            pl.BlockSpec((NC, blk, D), lambda i: (0, i, 0)),
            pl.BlockSpec((blk, 8), lambda i: (i, 0)),
            pl.BlockSpec((D, D), lambda i: (0, 0)),
            pl.BlockSpec((1, D), lambda i: (0, 0)),
            pl.BlockSpec((D, 2 * D), lambda i: (0, 0)),
            pl.BlockSpec((1, 2 * D), lambda i: (0, 0)),
            pl.BlockSpec((2 * D, D), lambda i: (0, 0)),
            pl.BlockSpec((1, D), lambda i: (0, 0)),
            pl.BlockSpec((1, D), lambda i: (0, 0)),
            pl.BlockSpec((1, D), lambda i: (0, 0)),
        ],
        out_specs=pl.BlockSpec((blk, D), lambda i: (i, 0)),
        out_shape=jax.ShapeDtypeStruct((N, D), jnp.float32),
    )(h, aggp, aggp2, degp, p["m2W"], p["m2b"].reshape(1, D),
      p["u1W"], p["u1b"].reshape(1, 2 * D),
      p["u2W"], p["u2b"].reshape(1, D),
      p["lnW"].reshape(1, D), p["lnb"].reshape(1, D))


def _pool_body(h_ref, lnw_ref, lnb_ref, o_ref):
    h = h_ref[...]
    seg = POOLED - 1
    chunk = math.ceil(N / seg)
    rid = lax.broadcasted_iota(jnp.int32, (N, 1), 0)
    toks = [jnp.sum(h, axis=0, keepdims=True) * (1.0 / N)]
    for k in range(seg):
        s = k * chunk
        t = min(N, (k + 1) * chunk)
        msk = ((rid >= s) & (rid < t)).astype(jnp.float32)
        toks.append(jnp.sum(h * msk, axis=0, keepdims=True) * (1.0 / (t - s)))
    st = jnp.concatenate(toks, axis=0)
    mu = jnp.mean(st, axis=1, keepdims=True)
    var = jnp.mean((st - mu) ** 2, axis=1, keepdims=True)
    o_ref[...] = (st - mu) / jnp.sqrt(var + 1e-5) * lnw_ref[...] + lnb_ref[...]


def _pool(h, lnw, lnb):
    return pl.pallas_call(
        _pool_body,
        out_shape=jax.ShapeDtypeStruct((POOLED, D), jnp.float32),
    )(h, lnw.reshape(1, D), lnb.reshape(1, D))


@functools.lru_cache(maxsize=None)
def _sc_mesh():
    return plsc.VectorSubcoreMesh(
        core_axis_name="c", subcore_axis_name="s",
        num_cores=NC, num_subcores=NS)


@functools.lru_cache(maxsize=None)
def _sc_gather_k(ep):
    ew = ep // NW
    nch = ew // CH

    @functools.partial(
        pl.kernel,
        out_type=jax.ShapeDtypeStruct((ep, D), jnp.float32),
        mesh=_sc_mesh(),
        scratch_types=[
            pltpu.VMEM((ew,), jnp.int32),
            pltpu.VMEM((3, CH, D), jnp.float32),
            pltpu.SemaphoreType.DMA,
            pltpu.SemaphoreType.DMA,
            pltpu.SemaphoreType.DMA,
        ],
    )
    def sc_gather(h_hbm, idx_hbm, out_hbm, idx_v, buf_v, sem0, sem1, sem2):
        wid = lax.axis_index("s") * NC + lax.axis_index("c")
        base = wid * ew
        sems = (sem0, sem1, sem2)
        pltpu.sync_copy(idx_hbm.at[pl.ds(base, ew)], idx_v)

        def fire(g, b):
            pltpu.async_copy(h_hbm.at[idx_v.at[pl.ds(g * CH, CH)]],
                             buf_v.at[b], sems[b])

        def wait(g, b):
            pltpu.make_async_copy(h_hbm.at[idx_v.at[pl.ds(g * CH, CH)]],
                                  buf_v.at[b], sems[b]).wait()

        for j in range(3):
            fire(j, j)

        @pl.loop(0, nch, step=3)
        def _body(g):
            for j in range(3):
                @pl.when(g + j < nch)
                def _do(j=j):
                    wait(g + j, j)
                    pltpu.sync_copy(
                        buf_v.at[j],
                        out_hbm.at[pl.ds(base + (g + j) * CH, CH)])

                    @pl.when(g + j + 3 < nch)
                    def _f():
                        fire(g + j + 3, j)

    return sc_gather


@functools.lru_cache(maxsize=None)
def _sc_scatter_k(ep):
    ew = ep // NW
    nch = ew // CH

    @functools.partial(
        pl.kernel,
        out_type=jax.ShapeDtypeStruct((NC, N, D), jnp.float32),
        mesh=_sc_mesh(),
        scratch_types=[
            pltpu.VMEM((nch, CH), jnp.int32),
            pltpu.VMEM((3, CH, D), jnp.float32),
            pltpu.VMEM_SHARED((N, D), jnp.float32),
            pltpu.SemaphoreType.DMA,
            pltpu.SemaphoreType.DMA,
            pltpu.SemaphoreType.DMA,
        ],
    )
    def sc_scatter(m_hbm, dst_hbm, out_hbm,
                   idx_v, buf_v, acc_sh, sem0, sem1, sem2):
        c = lax.axis_index("c")
        s = lax.axis_index("s")
        wid = s * NC + c
        base = wid * ew
        sems = (sem0, sem1, sem2)

        def fire(g, b):
            pltpu.async_copy(m_hbm.at[pl.ds(base + g * CH, CH)],
                             buf_v.at[b], sems[b])

        def wait(g, b):
            pltpu.make_async_copy(m_hbm.at[pl.ds(base + g * CH, CH)],
                                  buf_v.at[b], sems[b]).wait()

        fire(0, 0)
        fire(1, 1)

        zeros16 = jnp.zeros((16,), jnp.float32)

        @pl.loop(0, CH)
        def _zrow(r):
            for cc in range(D // 16):
                buf_v[2, r, pl.ds(cc * 16, 16)] = zeros16

        for k in range(SUB_ROWS // CH):
            pltpu.sync_copy(buf_v.at[2],
                            acc_sh.at[pl.ds(s * SUB_ROWS + k * CH, CH)])
        _zrem = SUB_ROWS - (SUB_ROWS // CH) * CH
        if _zrem:
            pltpu.sync_copy(
                buf_v.at[2].at[pl.ds(0, _zrem)],
                acc_sh.at[pl.ds(s * SUB_ROWS + (SUB_ROWS // CH) * CH, _zrem)])

        @pl.when(s == NS - 1)
        def _ztail():
            pltpu.sync_copy(buf_v.at[2].at[pl.ds(0, SUB_TAIL)],
                            acc_sh.at[pl.ds(NS * SUB_ROWS, SUB_TAIL)])

        pltpu.sync_copy(dst_hbm.at[wid], idx_v)
        fire(2, 2)
        plsc.subcore_barrier()

        @pl.loop(0, nch, step=3)
        def _body(g):
            for j in range(3):
                @pl.when(g + j < nch)
                def _do(j=j):
                    wait(g + j, j)
                    pltpu.sync_copy(buf_v.at[j], acc_sh.at[idx_v.at[g + j]],
                                    add=True)

                    @pl.when(g + j + 3 < nch)
                    def _f():
                        fire(g + j + 3, j)

        plsc.subcore_barrier()
        pltpu.sync_copy(acc_sh.at[pl.ds(s * SUB_ROWS, SUB_ROWS)],
                        out_hbm.at[c, pl.ds(s * SUB_ROWS, SUB_ROWS), :])

        @pl.when(s == NS - 1)
        def _otail():
            pltpu.sync_copy(acc_sh.at[pl.ds(NS * SUB_ROWS, SUB_TAIL)],
                            out_hbm.at[c, pl.ds(NS * SUB_ROWS, SUB_TAIL), :])

    return sc_scatter


@functools.lru_cache(maxsize=None)
def _sc_degree_k():
    ew = E // NW
    nch = ew // CH

    @functools.partial(
        pl.kernel,
        out_type=jax.ShapeDtypeStruct((NC, N, D), jnp.float32),
        mesh=_sc_mesh(),
        scratch_types=[
            pltpu.VMEM((nch, CH), jnp.int32),
            pltpu.VMEM((2, CH, D), jnp.float32),
            pltpu.VMEM_SHARED((N, D), jnp.float32),
            pltpu.SemaphoreType.DMA,
        ],
    )
    def sc_degree(dst_hbm, out_hbm, idx_v, buf_v, acc_sh, dsem):
        c = lax.axis_index("c")
        s = lax.axis_index("s")
        wid = s * NC + c
        ones_v = buf_v.at[0]
        ones16 = jnp.ones((16,), jnp.float32)
        zeros16 = jnp.zeros((16,), jnp.float32)

        @pl.loop(0, CH)
        def _fill(r):
            for cc in range(D // 16):
                buf_v[0, r, pl.ds(cc * 16, 16)] = ones16
                buf_v[1, r, pl.ds(cc * 16, 16)] = zeros16

        for k in range(SUB_ROWS // CH):
            pltpu.sync_copy(buf_v.at[1],
                            acc_sh.at[pl.ds(s * SUB_ROWS + k * CH, CH)])
        _zrem = SUB_ROWS - (SUB_ROWS // CH) * CH
        if _zrem:
            pltpu.sync_copy(
                buf_v.at[1].at[pl.ds(0, _zrem)],
                acc_sh.at[pl.ds(s * SUB_ROWS + (SUB_ROWS // CH) * CH, _zrem)])

        @pl.when(s == NS - 1)
        def _ztail():
            pltpu.sync_copy(buf_v.at[1].at[pl.ds(0, SUB_TAIL)],
                            acc_sh.at[pl.ds(NS * SUB_ROWS, SUB_TAIL)])

        pltpu.sync_copy(dst_hbm.at[wid], idx_v)
        plsc.subcore_barrier()

        @pl.loop(0, nch, step=5)
        def _chunk(g):
            for j in range(5):
                pltpu.async_copy(ones_v, acc_sh.at[idx_v.at[g + j]], dsem,
                                 add=True)
            for j in range(5):
                pltpu.make_async_copy(ones_v, acc_sh.at[idx_v.at[g + j]],
                                      dsem).wait()

        plsc.subcore_barrier()
        pltpu.sync_copy(acc_sh.at[pl.ds(s * SUB_ROWS, SUB_ROWS)],
                        out_hbm.at[c, pl.ds(s * SUB_ROWS, SUB_ROWS), :])

        @pl.when(s == NS - 1)
        def _otail():
            pltpu.sync_copy(acc_sh.at[pl.ds(NS * SUB_ROWS, SUB_TAIL)],
                            out_hbm.at[c, pl.ds(NS * SUB_ROWS, SUB_TAIL), :])

    return sc_degree


def _sc_gather(h, src_part):
    return _sc_gather_k(src_part.shape[0])(h, src_part)


def _sc_scatter(m, dst_part_r):
    ep = dst_part_r.shape[0] * dst_part_r.shape[1] * dst_part_r.shape[2]
    return _sc_scatter_k(ep)(m, dst_part_r)


def _sc_degree(dst_r):
    return _sc_degree_k()(dst_r)


def kernel(x, edge_index, edge_attr, params):
    src = edge_index[0]
    dst = edge_index[1]
    dst_r = dst.reshape(NW, (E // NW) // CH, CH)

    src_parts = [lax.slice(src, (o,), (o + p,)) for o, p in zip(POFF, PARTS)]
    dst_parts = [
        lax.slice(dst, (o,), (o + p,)).reshape(NW, (p // NW) // CH, CH)
        for o, p in zip(POFF, PARTS)
    ]

    h = _mm_bias(x, params["node_W"], params["node_b"], blk=2000)
    e = _mm_bias(edge_attr, params["edge_W"], params["edge_b"], blk=4000)
    degp = _deg_reduce(_sc_degree(dst_r))

    for i in range(3):
        p = params["l%d" % i]
        hs0 = _sc_gather(h, src_parts[0])
        g10 = _edge_mlp(hs0, e, p["m1W"], p["m1b"], POFF[0] // MLP_BLK)
        hs1 = _sc_gather(h, src_parts[1])
        g11 = _edge_mlp(hs1, e, p["m1W"], p["m1b"], POFF[1] // MLP_BLK)
        aggp0 = _sc_scatter(g10, dst_parts[0])
        aggp1 = _sc_scatter(g11, dst_parts[1])
        h = _node_update(h, aggp0, aggp1, degp, p)

    return _pool(h, params["out_lnW"], params["out_lnb"])


# speedup vs baseline: 1.0001x; 1.0001x over previous
import functools
import math

import jax
import jax.numpy as jnp
from jax import lax
from jax.experimental import pallas as pl
from jax.experimental.pallas import tpu as pltpu
from jax.experimental.pallas import tpu_sc as plsc

N = 10000
E = 320000
D = 128
D_EDGE = 16
POOLED = 8

NC = 2
NS = 16
NW = NC * NS
CH = 80
SUB_ROWS = 624
SUB_TAIL = N - NS * SUB_ROWS

PARTS = (192000, 128000)
POFF = (0, PARTS[0])

MLP_BLK = 4000


def _gelu(t):
    return 0.5 * t * (1.0 + lax.erf(t * (1.0 / math.sqrt(2.0))))


def _mm_bias_body(x_ref, w_ref, b_ref, o_ref):
    o_ref[...] = (
        jnp.dot(x_ref[...], w_ref[...], preferred_element_type=jnp.float32)
        + b_ref[...]
    )


def _mm_bias(x, w, b, blk):
    n, k = x.shape
    m = w.shape[1]
    return pl.pallas_call(
        _mm_bias_body,
        grid=(n // blk,),
        in_specs=[
            pl.BlockSpec((blk, k), lambda i: (i, 0)),
            pl.BlockSpec((k, m), lambda i: (0, 0)),
            pl.BlockSpec((1, m), lambda i: (0, 0)),
        ],
        out_specs=pl.BlockSpec((blk, m), lambda i: (i, 0)),
        out_shape=jax.ShapeDtypeStruct((n, m), jnp.float32),
    )(x, w, b.reshape(1, m))


def _edge_mlp_body(hs_ref, e_ref, w1_ref, b1_ref, o_ref):
    t = (
        jnp.dot(hs_ref[...] + e_ref[...], w1_ref[...],
                preferred_element_type=jnp.float32)
        + b1_ref[...]
    )
    o_ref[...] = _gelu(t)


def _edge_mlp(hs, e, w1, b1, e_off_blocks):
    ep = hs.shape[0]
    return pl.pallas_call(
        _edge_mlp_body,
        grid=(ep // MLP_BLK,),
        in_specs=[
            pl.BlockSpec((MLP_BLK, D), lambda i: (i, 0)),
            pl.BlockSpec((MLP_BLK, D), lambda i: (i + e_off_blocks, 0)),
            pl.BlockSpec((D, D), lambda i: (0, 0)),
            pl.BlockSpec((1, D), lambda i: (0, 0)),
        ],
        out_specs=pl.BlockSpec((MLP_BLK, D), lambda i: (i, 0)),
        out_shape=jax.ShapeDtypeStruct((ep, D), jnp.float32),
    )(hs, e, w1, b1.reshape(1, D))


def _deg_reduce_body(degp_ref, o_ref):
    deg = degp_ref[0, :, 0:1] + degp_ref[1, :, 0:1]
    o_ref[...] = jnp.broadcast_to(deg, (N, 8))


def _deg_reduce(degp):
    return pl.pallas_call(
        _deg_reduce_body,
        out_shape=jax.ShapeDtypeStruct((N, 8), jnp.float32),
    )(degp)


def _node_update_body(h_ref, agg_ref, agg2_ref, degp_ref, w2_ref, b2_ref,
                      u1w_ref, u1b_ref, u2w_ref, u2b_ref,
                      lnw_ref, lnb_ref, o_ref):
    agg1 = (agg_ref[0] + agg_ref[1]) + (agg2_ref[0] + agg2_ref[1])
    deg = degp_ref[:, 0:1]
    a2 = (
        jnp.dot(agg1, w2_ref[...], preferred_element_type=jnp.float32)
        + deg * b2_ref[...]
    )
    a2 = a2 / jnp.maximum(deg, 1.0)
    u = _gelu(
        jnp.dot(a2, u1w_ref[...], preferred_element_type=jnp.float32)
        + u1b_ref[...]
    )
    u = (
        jnp.dot(u, u2w_ref[...], preferred_element_type=jnp.float32)
        + u2b_ref[...]
    )
    hn = h_ref[...] + u
    mu = jnp.mean(hn, axis=1, keepdims=True)
    var = jnp.mean((hn - mu) ** 2, axis=1, keepdims=True)
    o_ref[...] = (hn - mu) / jnp.sqrt(var + 1e-5) * lnw_ref[...] + lnb_ref[...]


def _node_update(h, aggp, aggp2, degp, p, blk=2000):
    return pl.pallas_call(
        _node_update_body,
        grid=(N // blk,),
        in_specs=[
            pl.BlockSpec((blk, D), lambda i: (i, 0)),
            pl.BlockSpec((NC, blk, D), lambda i: (0, i, 0)),
            pl.BlockSpec((NC, blk, D), lambda i: (0, i, 0)),
            pl.BlockSpec((blk, 8), lambda i: (i, 0)),
            pl.BlockSpec((D, D), lambda i: (0, 0)),
            pl.BlockSpec((1, D), lambda i: (0, 0)),
            pl.BlockSpec((D, 2 * D), lambda i: (0, 0)),
            pl.BlockSpec((1, 2 * D), lambda i: (0, 0)),
            pl.BlockSpec((2 * D, D), lambda i: (0, 0)),
            pl.BlockSpec((1, D), lambda i: (0, 0)),
            pl.BlockSpec((1, D), lambda i: (0, 0)),
            pl.BlockSpec((1, D), lambda i: (0, 0)),
        ],
        out_specs=pl.BlockSpec((blk, D), lambda i: (i, 0)),
        out_shape=jax.ShapeDtypeStruct((N, D), jnp.float32),
    )(h, aggp, aggp2, degp, p["m2W"], p["m2b"].reshape(1, D),
      p["u1W"], p["u1b"].reshape(1, 2 * D),
      p["u2W"], p["u2b"].reshape(1, D),
      p["lnW"].reshape(1, D), p["lnb"].reshape(1, D))


def _pool_body(h_ref, lnw_ref, lnb_ref, o_ref):
    h = h_ref[...]
    seg = POOLED - 1
    chunk = math.ceil(N / seg)
    rid = lax.broadcasted_iota(jnp.int32, (N, 1), 0)
    toks = [jnp.sum(h, axis=0, keepdims=True) * (1.0 / N)]
    for k in range(seg):
        s = k * chunk
        t = min(N, (k + 1) * chunk)
        msk = ((rid >= s) & (rid < t)).astype(jnp.float32)
        toks.append(jnp.sum(h * msk, axis=0, keepdims=True) * (1.0 / (t - s)))
    st = jnp.concatenate(toks, axis=0)
    mu = jnp.mean(st, axis=1, keepdims=True)
    var = jnp.mean((st - mu) ** 2, axis=1, keepdims=True)
    o_ref[...] = (st - mu) / jnp.sqrt(var + 1e-5) * lnw_ref[...] + lnb_ref[...]


def _pool(h, lnw, lnb):
    return pl.pallas_call(
        _pool_body,
        out_shape=jax.ShapeDtypeStruct((POOLED, D), jnp.float32),
    )(h, lnw.reshape(1, D), lnb.reshape(1, D))


@functools.lru_cache(maxsize=None)
def _sc_mesh():
    return plsc.VectorSubcoreMesh(
        core_axis_name="c", subcore_axis_name="s",
        num_cores=NC, num_subcores=NS)


@functools.lru_cache(maxsize=None)
def _sc_gather_k(ep):
    ew = ep // NW
    nch = ew // CH

    @functools.partial(
        pl.kernel,
        out_type=jax.ShapeDtypeStruct((ep, D), jnp.float32),
        mesh=_sc_mesh(),
        scratch_types=[
            pltpu.VMEM((ew,), jnp.int32),
            pltpu.VMEM((3, CH, D), jnp.float32),
            pltpu.SemaphoreType.DMA,
            pltpu.SemaphoreType.DMA,
            pltpu.SemaphoreType.DMA,
        ],
    )
    def sc_gather(h_hbm, idx_hbm, out_hbm, idx_v, buf_v, sem0, sem1, sem2):
        wid = lax.axis_index("s") * NC + lax.axis_index("c")
        base = wid * ew
        sems = (sem0, sem1, sem2)
        pltpu.sync_copy(idx_hbm.at[pl.ds(base, ew)], idx_v)

        def fire(g, b):
            pltpu.async_copy(h_hbm.at[idx_v.at[pl.ds(g * CH, CH)]],
                             buf_v.at[b], sems[b])

        def wait(g, b):
            pltpu.make_async_copy(h_hbm.at[idx_v.at[pl.ds(g * CH, CH)]],
                                  buf_v.at[b], sems[b]).wait()

        for j in range(3):
            fire(j, j)

        @pl.loop(0, nch, step=3)
        def _body(g):
            for j in range(3):
                @pl.when(g + j < nch)
                def _do(j=j):
                    wait(g + j, j)
                    pltpu.sync_copy(
                        buf_v.at[j],
                        out_hbm.at[pl.ds(base + (g + j) * CH, CH)])

                    @pl.when(g + j + 3 < nch)
                    def _f():
                        fire(g + j + 3, j)

    return sc_gather


@functools.lru_cache(maxsize=None)
def _sc_scatter_k(ep):
    ew = ep // NW
    nch = ew // CH

    @functools.partial(
        pl.kernel,
        out_type=jax.ShapeDtypeStruct((NC, N, D), jnp.float32),
        mesh=_sc_mesh(),
        scratch_types=[
            pltpu.VMEM((nch, CH), jnp.int32),
            pltpu.VMEM((3, CH, D), jnp.float32),
            pltpu.VMEM_SHARED((N, D), jnp.float32),
            pltpu.SemaphoreType.DMA,
            pltpu.SemaphoreType.DMA,
            pltpu.SemaphoreType.DMA,
        ],
    )
    def sc_scatter(m_hbm, dst_hbm, out_hbm,
                   idx_v, buf_v, acc_sh, sem0, sem1, sem2):
        c = lax.axis_index("c")
        s = lax.axis_index("s")
        wid = s * NC + c
        base = wid * ew
        sems = (sem0, sem1, sem2)

        def fire(g, b):
            pltpu.async_copy(m_hbm.at[pl.ds(base + g * CH, CH)],
                             buf_v.at[b], sems[b])

        def wait(g, b):
            pltpu.make_async_copy(m_hbm.at[pl.ds(base + g * CH, CH)],
                                  buf_v.at[b], sems[b]).wait()

        fire(0, 0)
        fire(1, 1)

        zeros16 = jnp.zeros((16,), jnp.float32)

        @pl.loop(0, CH)
        def _zrow(r):
            for cc in range(D // 16):
                buf_v[2, r, pl.ds(cc * 16, 16)] = zeros16

        for k in range(SUB_ROWS // CH):
            pltpu.sync_copy(buf_v.at[2],
                            acc_sh.at[pl.ds(s * SUB_ROWS + k * CH, CH)])
        _zrem = SUB_ROWS - (SUB_ROWS // CH) * CH
        if _zrem:
            pltpu.sync_copy(
                buf_v.at[2].at[pl.ds(0, _zrem)],
                acc_sh.at[pl.ds(s * SUB_ROWS + (SUB_ROWS // CH) * CH, _zrem)])

        @pl.when(s == NS - 1)
        def _ztail():
            pltpu.sync_copy(buf_v.at[2].at[pl.ds(0, SUB_TAIL)],
                            acc_sh.at[pl.ds(NS * SUB_ROWS, SUB_TAIL)])

        pltpu.sync_copy(dst_hbm.at[wid], idx_v)
        fire(2, 2)
        plsc.subcore_barrier()

        @pl.loop(0, nch, step=3)
        def _body(g):
            for j in range(3):
                @pl.when(g + j < nch)
                def _do(j=j):
                    wait(g + j, j)
                    pltpu.sync_copy(buf_v.at[j], acc_sh.at[idx_v.at[g + j]],
                                    add=True)

                    @pl.when(g + j + 3 < nch)
                    def _f():
                        fire(g + j + 3, j)

        plsc.subcore_barrier()
        pltpu.sync_copy(acc_sh.at[pl.ds(s * SUB_ROWS, SUB_ROWS)],
                        out_hbm.at[c, pl.ds(s * SUB_ROWS, SUB_ROWS), :])

        @pl.when(s == NS - 1)
        def _otail():
            pltpu.sync_copy(acc_sh.at[pl.ds(NS * SUB_ROWS, SUB_TAIL)],
                            out_hbm.at[c, pl.ds(NS * SUB_ROWS, SUB_TAIL), :])

    return sc_scatter


@functools.lru_cache(maxsize=None)
def _sc_degree_k():
    ew = E // NW
    nch = ew // CH

    @functools.partial(
        pl.kernel,
        out_type=jax.ShapeDtypeStruct((NC, N, D), jnp.float32),
        mesh=_sc_mesh(),
        scratch_types=[
            pltpu.VMEM((nch, CH), jnp.int32),
            pltpu.VMEM((2, CH, D), jnp.float32),
            pltpu.VMEM_SHARED((N, D), jnp.float32),
            pltpu.SemaphoreType.DMA,
        ],
    )
    def sc_degree(dst_hbm, out_hbm, idx_v, buf_v, acc_sh, dsem):
        c = lax.axis_index("c")
        s = lax.axis_index("s")
        wid = s * NC + c
        ones_v = buf_v.at[0]
        ones16 = jnp.ones((16,), jnp.float32)
        zeros16 = jnp.zeros((16,), jnp.float32)

        @pl.loop(0, CH)
        def _fill(r):
            for cc in range(D // 16):
                buf_v[0, r, pl.ds(cc * 16, 16)] = ones16
                buf_v[1, r, pl.ds(cc * 16, 16)] = zeros16

        for k in range(SUB_ROWS // CH):
            pltpu.sync_copy(buf_v.at[1],
                            acc_sh.at[pl.ds(s * SUB_ROWS + k * CH, CH)])
        _zrem = SUB_ROWS - (SUB_ROWS // CH) * CH
        if _zrem:
            pltpu.sync_copy(
                buf_v.at[1].at[pl.ds(0, _zrem)],
                acc_sh.at[pl.ds(s * SUB_ROWS + (SUB_ROWS // CH) * CH, _zrem)])

        @pl.when(s == NS - 1)
        def _ztail():
            pltpu.sync_copy(buf_v.at[1].at[pl.ds(0, SUB_TAIL)],
                            acc_sh.at[pl.ds(NS * SUB_ROWS, SUB_TAIL)])

        pltpu.sync_copy(dst_hbm.at[wid], idx_v)
        plsc.subcore_barrier()

        @pl.loop(0, nch, step=5)
        def _chunk(g):
            for j in range(5):
                pltpu.async_copy(ones_v, acc_sh.at[idx_v.at[g + j]], dsem,
                                 add=True)
            for j in range(5):
                pltpu.make_async_copy(ones_v, acc_sh.at[idx_v.at[g + j]],
                                      dsem).wait()

        plsc.subcore_barrier()
        pltpu.sync_copy(acc_sh.at[pl.ds(s * SUB_ROWS, SUB_ROWS)],
                        out_hbm.at[c, pl.ds(s * SUB_ROWS, SUB_ROWS), :])

        @pl.when(s == NS - 1)
        def _otail():
            pltpu.sync_copy(acc_sh.at[pl.ds(NS * SUB_ROWS, SUB_TAIL)],
                            out_hbm.at[c, pl.ds(NS * SUB_ROWS, SUB_TAIL), :])

    return sc_degree


def _sc_gather(h, src_part):
    return _sc_gather_k(src_part.shape[0])(h, src_part)


def _sc_scatter(m, dst_part_r):
    ep = dst_part_r.shape[0] * dst_part_r.shape[1] * dst_part_r.shape[2]
    return _sc_scatter_k(ep)(m, dst_part_r)


def _sc_degree(dst_r):
    return _sc_degree_k()(dst_r)


def kernel(x, edge_index, edge_attr, params):
    src = edge_index[0]
    dst = edge_index[1]
    dst_r = dst.reshape(NW, (E // NW) // CH, CH)

    src_parts = [lax.slice(src, (o,), (o + p,)) for o, p in zip(POFF, PARTS)]
    dst_parts = [
        lax.slice(dst, (o,), (o + p,)).reshape(NW, (p // NW) // CH, CH)
        for o, p in zip(POFF, PARTS)
    ]

    h = _mm_bias(x, params["node_W"], params["node_b"], blk=2000)
    e = _mm_bias(edge_attr, params["edge_W"], params["edge_b"], blk=4000)
    degp = _deg_reduce(_sc_degree(dst_r))

    for i in range(3):
        p = params["l%d" % i]
        hs0 = _sc_gather(h, src_parts[0])
        hs1 = _sc_gather(h, src_parts[1])
        g10 = _edge_mlp(hs0, e, p["m1W"], p["m1b"], POFF[0] // MLP_BLK)
        aggp0 = _sc_scatter(g10, dst_parts[0])
        g11 = _edge_mlp(hs1, e, p["m1W"], p["m1b"], POFF[1] // MLP_BLK)
        aggp1 = _sc_scatter(g11, dst_parts[1])
        h = _node_update(h, aggp0, aggp1, degp, p)

    return _pool(h, params["out_lnW"], params["out_lnb"])


# speedup vs baseline: 1.0590x; 1.0589x over previous
import functools
import math

import jax
import jax.numpy as jnp
from jax import lax
from jax.experimental import pallas as pl
from jax.experimental.pallas import tpu as pltpu
from jax.experimental.pallas import tpu_sc as plsc

N = 10000
E = 320000
D = 128
D_EDGE = 16
POOLED = 8

NC = 2
NS = 16
NW = NC * NS
CH = 80
SUB_ROWS = 624
SUB_TAIL = N - NS * SUB_ROWS

PARTS = (192000, 128000)
POFF = (0, PARTS[0])

MLP_BLK = 4000


def _gelu(t):
    return 0.5 * t * (1.0 + lax.erf(t * (1.0 / math.sqrt(2.0))))


def _mm_bias_body(x_ref, w_ref, b_ref, o_ref):
    o_ref[...] = (
        jnp.dot(x_ref[...], w_ref[...], preferred_element_type=jnp.float32)
        + b_ref[...]
    ).astype(o_ref.dtype)


def _mm_bias(x, w, b, blk, out_dtype=jnp.float32):
    n, k = x.shape
    m = w.shape[1]
    return pl.pallas_call(
        _mm_bias_body,
        grid=(n // blk,),
        in_specs=[
            pl.BlockSpec((blk, k), lambda i: (i, 0)),
            pl.BlockSpec((k, m), lambda i: (0, 0)),
            pl.BlockSpec((1, m), lambda i: (0, 0)),
        ],
        out_specs=pl.BlockSpec((blk, m), lambda i: (i, 0)),
        out_shape=jax.ShapeDtypeStruct((n, m), out_dtype),
    )(x, w, b.reshape(1, m))


def _edge_mlp_body(hs_ref, e_ref, w1_ref, b1_ref, o_ref):
    t = (
        jnp.dot(hs_ref[...] + e_ref[...].astype(jnp.float32), w1_ref[...],
                preferred_element_type=jnp.float32)
        + b1_ref[...]
    )
    o_ref[...] = _gelu(t)


def _edge_mlp(hs, e, w1, b1, e_off_blocks):
    ep = hs.shape[0]
    return pl.pallas_call(
        _edge_mlp_body,
        grid=(ep // MLP_BLK,),
        in_specs=[
            pl.BlockSpec((MLP_BLK, D), lambda i: (i, 0)),
            pl.BlockSpec((MLP_BLK, D), lambda i: (i + e_off_blocks, 0)),
            pl.BlockSpec((D, D), lambda i: (0, 0)),
            pl.BlockSpec((1, D), lambda i: (0, 0)),
        ],
        out_specs=pl.BlockSpec((MLP_BLK, D), lambda i: (i, 0)),
        out_shape=jax.ShapeDtypeStruct((ep, D), jnp.float32),
    )(hs, e, w1, b1.reshape(1, D))


def _deg_reduce_body(degp_ref, o_ref):
    deg = degp_ref[0, :, 0:1] + degp_ref[1, :, 0:1]
    o_ref[...] = jnp.broadcast_to(deg, (N, 8))


def _deg_reduce(degp):
    return pl.pallas_call(
        _deg_reduce_body,
        out_shape=jax.ShapeDtypeStruct((N, 8), jnp.float32),
    )(degp)


def _node_update_body(h_ref, agg_ref, agg2_ref, degp_ref, w2_ref, b2_ref,
                      u1w_ref, u1b_ref, u2w_ref, u2b_ref,
                      lnw_ref, lnb_ref, o_ref):
    agg1 = (agg_ref[0] + agg_ref[1]) + (agg2_ref[0] + agg2_ref[1])
    deg = degp_ref[:, 0:1]
    a2 = (
        jnp.dot(agg1, w2_ref[...], preferred_element_type=jnp.float32)
        + deg * b2_ref[...]
    )
    a2 = a2 / jnp.maximum(deg, 1.0)
    u = _gelu(
        jnp.dot(a2, u1w_ref[...], preferred_element_type=jnp.float32)
        + u1b_ref[...]
    )
    u = (
        jnp.dot(u, u2w_ref[...], preferred_element_type=jnp.float32)
        + u2b_ref[...]
    )
    hn = h_ref[...] + u
    mu = jnp.mean(hn, axis=1, keepdims=True)
    var = jnp.mean((hn - mu) ** 2, axis=1, keepdims=True)
    o_ref[...] = (hn - mu) / jnp.sqrt(var + 1e-5) * lnw_ref[...] + lnb_ref[...]


def _node_update(h, aggp, aggp2, degp, p, blk=2000):
    return pl.pallas_call(
        _node_update_body,
        grid=(N // blk,),
        in_specs=[
            pl.BlockSpec((blk, D), lambda i: (i, 0)),
            pl.BlockSpec((NC, blk, D), lambda i: (0, i, 0)),
            pl.BlockSpec((NC, blk, D), lambda i: (0, i, 0)),
            pl.BlockSpec((blk, 8), lambda i: (i, 0)),
            pl.BlockSpec((D, D), lambda i: (0, 0)),
            pl.BlockSpec((1, D), lambda i: (0, 0)),
            pl.BlockSpec((D, 2 * D), lambda i: (0, 0)),
            pl.BlockSpec((1, 2 * D), lambda i: (0, 0)),
            pl.BlockSpec((2 * D, D), lambda i: (0, 0)),
            pl.BlockSpec((1, D), lambda i: (0, 0)),
            pl.BlockSpec((1, D), lambda i: (0, 0)),
            pl.BlockSpec((1, D), lambda i: (0, 0)),
        ],
        out_specs=pl.BlockSpec((blk, D), lambda i: (i, 0)),
        out_shape=jax.ShapeDtypeStruct((N, D), jnp.float32),
    )(h, aggp, aggp2, degp, p["m2W"], p["m2b"].reshape(1, D),
      p["u1W"], p["u1b"].reshape(1, 2 * D),
      p["u2W"], p["u2b"].reshape(1, D),
      p["lnW"].reshape(1, D), p["lnb"].reshape(1, D))


def _pool_body(h_ref, lnw_ref, lnb_ref, o_ref):
    h = h_ref[...]
    seg = POOLED - 1
    chunk = math.ceil(N / seg)
    rid = lax.broadcasted_iota(jnp.int32, (N, 1), 0)
    toks = [jnp.sum(h, axis=0, keepdims=True) * (1.0 / N)]
    for k in range(seg):
        s = k * chunk
        t = min(N, (k + 1) * chunk)
        msk = ((rid >= s) & (rid < t)).astype(jnp.float32)
        toks.append(jnp.sum(h * msk, axis=0, keepdims=True) * (1.0 / (t - s)))
    st = jnp.concatenate(toks, axis=0)
    mu = jnp.mean(st, axis=1, keepdims=True)
    var = jnp.mean((st - mu) ** 2, axis=1, keepdims=True)
    o_ref[...] = (st - mu) / jnp.sqrt(var + 1e-5) * lnw_ref[...] + lnb_ref[...]


def _pool(h, lnw, lnb):
    return pl.pallas_call(
        _pool_body,
        out_shape=jax.ShapeDtypeStruct((POOLED, D), jnp.float32),
    )(h, lnw.reshape(1, D), lnb.reshape(1, D))


@functools.lru_cache(maxsize=None)
def _sc_mesh():
    return plsc.VectorSubcoreMesh(
        core_axis_name="c", subcore_axis_name="s",
        num_cores=NC, num_subcores=NS)


@functools.lru_cache(maxsize=None)
def _sc_gather_k(ep):
    ew = ep // NW
    nch = ew // CH

    @functools.partial(
        pl.kernel,
        out_type=jax.ShapeDtypeStruct((ep, D), jnp.float32),
        mesh=_sc_mesh(),
        scratch_types=[
            pltpu.VMEM((ew,), jnp.int32),
            pltpu.VMEM((3, CH, D), jnp.float32),
            pltpu.SemaphoreType.DMA,
            pltpu.SemaphoreType.DMA,
            pltpu.SemaphoreType.DMA,
        ],
    )
    def sc_gather(h_hbm, idx_hbm, out_hbm, idx_v, buf_v, sem0, sem1, sem2):
        wid = lax.axis_index("s") * NC + lax.axis_index("c")
        base = wid * ew
        sems = (sem0, sem1, sem2)
        pltpu.sync_copy(idx_hbm.at[pl.ds(base, ew)], idx_v)

        def fire(g, b):
            pltpu.async_copy(h_hbm.at[idx_v.at[pl.ds(g * CH, CH)]],
                             buf_v.at[b], sems[b])

        def wait(g, b):
            pltpu.make_async_copy(h_hbm.at[idx_v.at[pl.ds(g * CH, CH)]],
                                  buf_v.at[b], sems[b]).wait()

        for j in range(3):
            fire(j, j)

        @pl.loop(0, nch, step=3)
        def _body(g):
            for j in range(3):
                @pl.when(g + j < nch)
                def _do(j=j):
                    wait(g + j, j)
                    pltpu.sync_copy(
                        buf_v.at[j],
                        out_hbm.at[pl.ds(base + (g + j) * CH, CH)])

                    @pl.when(g + j + 3 < nch)
                    def _f():
                        fire(g + j + 3, j)

    return sc_gather


@functools.lru_cache(maxsize=None)
def _sc_scatter_k(ep):
    ew = ep // NW
    nch = ew // CH

    @functools.partial(
        pl.kernel,
        out_type=jax.ShapeDtypeStruct((NC, N, D), jnp.float32),
        mesh=_sc_mesh(),
        scratch_types=[
            pltpu.VMEM((nch, CH), jnp.int32),
            pltpu.VMEM((3, CH, D), jnp.float32),
            pltpu.VMEM_SHARED((N, D), jnp.float32),
            pltpu.SemaphoreType.DMA,
            pltpu.SemaphoreType.DMA,
            pltpu.SemaphoreType.DMA,
        ],
    )
    def sc_scatter(m_hbm, dst_hbm, out_hbm,
                   idx_v, buf_v, acc_sh, sem0, sem1, sem2):
        c = lax.axis_index("c")
        s = lax.axis_index("s")
        wid = s * NC + c
        base = wid * ew
        sems = (sem0, sem1, sem2)

        def fire(g, b):
            pltpu.async_copy(m_hbm.at[pl.ds(base + g * CH, CH)],
                             buf_v.at[b], sems[b])

        def wait(g, b):
            pltpu.make_async_copy(m_hbm.at[pl.ds(base + g * CH, CH)],
                                  buf_v.at[b], sems[b]).wait()

        fire(0, 0)
        fire(1, 1)

        zeros16 = jnp.zeros((16,), jnp.float32)

        @pl.loop(0, CH)
        def _zrow(r):
            for cc in range(D // 16):
                buf_v[2, r, pl.ds(cc * 16, 16)] = zeros16

        for k in range(SUB_ROWS // CH):
            pltpu.sync_copy(buf_v.at[2],
                            acc_sh.at[pl.ds(s * SUB_ROWS + k * CH, CH)])
        _zrem = SUB_ROWS - (SUB_ROWS // CH) * CH
        if _zrem:
            pltpu.sync_copy(
                buf_v.at[2].at[pl.ds(0, _zrem)],
                acc_sh.at[pl.ds(s * SUB_ROWS + (SUB_ROWS // CH) * CH, _zrem)])

        @pl.when(s == NS - 1)
        def _ztail():
            pltpu.sync_copy(buf_v.at[2].at[pl.ds(0, SUB_TAIL)],
                            acc_sh.at[pl.ds(NS * SUB_ROWS, SUB_TAIL)])

        pltpu.sync_copy(dst_hbm.at[wid], idx_v)
        fire(2, 2)
        plsc.subcore_barrier()

        @pl.loop(0, nch, step=3)
        def _body(g):
            for j in range(3):
                @pl.when(g + j < nch)
                def _do(j=j):
                    wait(g + j, j)
                    pltpu.sync_copy(buf_v.at[j], acc_sh.at[idx_v.at[g + j]],
                                    add=True)

                    @pl.when(g + j + 3 < nch)
                    def _f():
                        fire(g + j + 3, j)

        plsc.subcore_barrier()
        pltpu.sync_copy(acc_sh.at[pl.ds(s * SUB_ROWS, SUB_ROWS)],
                        out_hbm.at[c, pl.ds(s * SUB_ROWS, SUB_ROWS), :])

        @pl.when(s == NS - 1)
        def _otail():
            pltpu.sync_copy(acc_sh.at[pl.ds(NS * SUB_ROWS, SUB_TAIL)],
                            out_hbm.at[c, pl.ds(NS * SUB_ROWS, SUB_TAIL), :])

    return sc_scatter


@functools.lru_cache(maxsize=None)
def _sc_degree_k():
    ew = E // NW
    nch = ew // CH

    @functools.partial(
        pl.kernel,
        out_type=jax.ShapeDtypeStruct((NC, N, D), jnp.float32),
        mesh=_sc_mesh(),
        scratch_types=[
            pltpu.VMEM((nch, CH), jnp.int32),
            pltpu.VMEM((2, CH, D), jnp.float32),
            pltpu.VMEM_SHARED((N, D), jnp.float32),
            pltpu.SemaphoreType.DMA,
        ],
    )
    def sc_degree(dst_hbm, out_hbm, idx_v, buf_v, acc_sh, dsem):
        c = lax.axis_index("c")
        s = lax.axis_index("s")
        wid = s * NC + c
        ones_v = buf_v.at[0]
        ones16 = jnp.ones((16,), jnp.float32)
        zeros16 = jnp.zeros((16,), jnp.float32)

        @pl.loop(0, CH)
        def _fill(r):
            for cc in range(D // 16):
                buf_v[0, r, pl.ds(cc * 16, 16)] = ones16
                buf_v[1, r, pl.ds(cc * 16, 16)] = zeros16

        for k in range(SUB_ROWS // CH):
            pltpu.sync_copy(buf_v.at[1],
                            acc_sh.at[pl.ds(s * SUB_ROWS + k * CH, CH)])
        _zrem = SUB_ROWS - (SUB_ROWS // CH) * CH
        if _zrem:
            pltpu.sync_copy(
                buf_v.at[1].at[pl.ds(0, _zrem)],
                acc_sh.at[pl.ds(s * SUB_ROWS + (SUB_ROWS // CH) * CH, _zrem)])

        @pl.when(s == NS - 1)
        def _ztail():
            pltpu.sync_copy(buf_v.at[1].at[pl.ds(0, SUB_TAIL)],
                            acc_sh.at[pl.ds(NS * SUB_ROWS, SUB_TAIL)])

        pltpu.sync_copy(dst_hbm.at[wid], idx_v)
        plsc.subcore_barrier()

        @pl.loop(0, nch, step=5)
        def _chunk(g):
            for j in range(5):
                pltpu.async_copy(ones_v, acc_sh.at[idx_v.at[g + j]], dsem,
                                 add=True)
            for j in range(5):
                pltpu.make_async_copy(ones_v, acc_sh.at[idx_v.at[g + j]],
                                      dsem).wait()

        plsc.subcore_barrier()
        pltpu.sync_copy(acc_sh.at[pl.ds(s * SUB_ROWS, SUB_ROWS)],
                        out_hbm.at[c, pl.ds(s * SUB_ROWS, SUB_ROWS), :])

        @pl.when(s == NS - 1)
        def _otail():
            pltpu.sync_copy(acc_sh.at[pl.ds(NS * SUB_ROWS, SUB_TAIL)],
                            out_hbm.at[c, pl.ds(NS * SUB_ROWS, SUB_TAIL), :])

    return sc_degree


def _sc_gather(h, src_part):
    return _sc_gather_k(src_part.shape[0])(h, src_part)


def _sc_scatter(m, dst_part_r):
    ep = dst_part_r.shape[0] * dst_part_r.shape[1] * dst_part_r.shape[2]
    return _sc_scatter_k(ep)(m, dst_part_r)


def _sc_degree(dst_r):
    return _sc_degree_k()(dst_r)


def kernel(x, edge_index, edge_attr, params):
    src = edge_index[0]
    dst = edge_index[1]
    dst_r = dst.reshape(NW, (E // NW) // CH, CH)

    src_parts = [lax.slice(src, (o,), (o + p,)) for o, p in zip(POFF, PARTS)]
    dst_parts = [
        lax.slice(dst, (o,), (o + p,)).reshape(NW, (p // NW) // CH, CH)
        for o, p in zip(POFF, PARTS)
    ]

    h = _mm_bias(x, params["node_W"], params["node_b"], blk=2000)
    e = _mm_bias(edge_attr, params["edge_W"], params["edge_b"], blk=4000,
                 out_dtype=jnp.bfloat16)
    degp = _deg_reduce(_sc_degree(dst_r))

    for i in range(3):
        p = params["l%d" % i]
        hs0 = _sc_gather(h, src_parts[0])
        hs1 = _sc_gather(h, src_parts[1])
        g10 = _edge_mlp(hs0, e, p["m1W"], p["m1b"], POFF[0] // MLP_BLK)
        aggp0 = _sc_scatter(g10, dst_parts[0])
        g11 = _edge_mlp(hs1, e, p["m1W"], p["m1b"], POFF[1] // MLP_BLK)
        aggp1 = _sc_scatter(g11, dst_parts[1])
        h = _node_update(h, aggp0, aggp1, degp, p)

    return _pool(h, params["out_lnW"], params["out_lnb"])
